# baseline (speedup 1.0000x reference)
import functools
import jax
import jax.numpy as jnp
from jax import lax
from jax.experimental import pallas as pl
from jax.experimental.pallas import tpu as pltpu
from jax.experimental.pallas import tpu_sc as plsc

NC = 2
NS = 16
NW = NC * NS
L = 16

VOCAB = 100000
D = 64
SEQ = 200
BATCH = 4096

D_PER_W = D // NW
GROUPS = BATCH // L
G_IN = 8


def _splat(val):
    return jax.lax.broadcast_in_dim(val, (L,), ())


def _body(xt, wt, post, out, row_v, posr_v, ib0, ib1, ob0, ob1, ob2, ob3,
          si0, si1, so0, so1, so2, so3):
    w = lax.axis_index("s") * NC + lax.axis_index("c")
    ibufs, sis = (ib0, ib1), (si0, si1)
    obufs, sos = (ob0, ob1, ob2, ob3), (so0, so1, so2, so3)

    def out_slice(s, dt, di):
        return out.at[s, dt, pl.ds(0, 32), pl.ds(di, 1), pl.ds(0, 128)]

    def phase(dn, _):
        d = w * D_PER_W + dn
        dt = d // 8
        di = d % 8
        pltpu.sync_copy(wt.at[d], row_v)
        pltpu.sync_copy(post.at[d], posr_v)
        pltpu.async_copy(xt.at[0], ib0, si0)

        def sbody(k, _):
            for q in range(4):
                s = 4 * k + q
                p = q % 2
                ibuf, obuf = ibufs[p], obufs[q]
                pltpu.make_async_copy(xt.at[s], ibuf, sis[p]).wait()
                @pl.when(k > 0)
                def _():
                    pltpu.make_async_copy(
                        obuf, out_slice(s, dt, di), sos[q]
                    ).wait()

                qp = (q - 1) % 4

                def _ship():
                    pltpu.async_copy(
                        obufs[qp], out_slice(s - 1, dt, di), sos[qp]
                    )

                if q == 0:
                    pl.when(k > 0)(_ship)
                else:
                    _ship()

                @pl.when(s + 1 < SEQ)
                def _():
                    pltpu.async_copy(xt.at[s + 1], ibufs[1 - p], sis[1 - p])

                pv = plsc.load_gather(posr_v, [_splat(s)])

                @plsc.parallel_loop(0, GROUPS // G_IN, 1, unroll=8)
                def _(go):
                    base = go * (G_IN * L)
                    for gi in range(G_IN):
                        iv = ibuf[pl.ds(base + gi * L, L)]
                        gv = plsc.load_gather(row_v, [iv])
                        obuf[go, 0, pl.ds(gi * L, L)] = gv + pv

            return ()

        lax.fori_loop(0, SEQ // 4, sbody, ())
        for q in range(3):
            pltpu.make_async_copy(
                obufs[q], out_slice(0, 0, 0), sos[q]
            ).wait()
        pltpu.async_copy(obufs[3], out_slice(SEQ - 1, dt, di), sos[3])
        pltpu.make_async_copy(obufs[3], out_slice(0, 0, 0), sos[3]).wait()
        return ()

    lax.fori_loop(0, D_PER_W, phase, ())


@jax.jit
def kernel(x, word_table, pos_table):
    xt = x.T
    wt = word_table.T
    post = pos_table.T
    mesh = plsc.VectorSubcoreMesh(core_axis_name="c", subcore_axis_name="s")
    out5 = pl.kernel(
        _body,
        out_type=jax.ShapeDtypeStruct((SEQ, 8, 32, 8, 128), jnp.float32),
        mesh=mesh,
        compiler_params=pltpu.CompilerParams(
            use_tc_tiling_on_sc=False, needs_layout_passes=False
        ),
        scratch_types=[
            pltpu.VMEM((VOCAB,), jnp.float32),
            pltpu.VMEM((SEQ,), jnp.float32),
            pltpu.VMEM((BATCH,), jnp.int32),
            pltpu.VMEM((BATCH,), jnp.int32),
            pltpu.VMEM((32, 1, 128), jnp.float32),
            pltpu.VMEM((32, 1, 128), jnp.float32),
            pltpu.VMEM((32, 1, 128), jnp.float32),
            pltpu.VMEM((32, 1, 128), jnp.float32),
            pltpu.SemaphoreType.DMA,
            pltpu.SemaphoreType.DMA,
            pltpu.SemaphoreType.DMA,
            pltpu.SemaphoreType.DMA,
            pltpu.SemaphoreType.DMA,
            pltpu.SemaphoreType.DMA,
        ],
    )(xt, wt, post)
    return out5.transpose(2, 4, 0, 1, 3).reshape(BATCH, SEQ, D)


# speedup vs baseline: 1.0010x; 1.0010x over previous
import functools
import jax
import jax.numpy as jnp
from jax import lax
from jax.experimental import pallas as pl
from jax.experimental.pallas import tpu as pltpu
from jax.experimental.pallas import tpu_sc as plsc

NC = 2
NS = 16
NW = NC * NS
L = 16

VOCAB = 100000
D = 64
SEQ = 200
BATCH = 4096

D_PER_W = D // NW
GROUPS = BATCH // L
G_IN = 8


def _splat(val):
    return jax.lax.broadcast_in_dim(val, (L,), ())


def _body(xt, wt, post, out, row_v, posr_v, ib0, ib1, ob0, ob1, ob2, ob3,
          si0, si1, so0, so1, so2, so3):
    w = lax.axis_index("s") * NC + lax.axis_index("c")
    ibufs, sis = (ib0, ib1), (si0, si1)
    obufs, sos = (ob0, ob1, ob2, ob3), (so0, so1, so2, so3)

    def out_slice(s, dt, di):
        return out.at[s, dt, pl.ds(0, 32), pl.ds(di, 1), pl.ds(0, 128)]

    def phase(dn, _):
        d = w * D_PER_W + dn
        dt = d // 8
        di = d % 8
        pltpu.sync_copy(wt.at[d], row_v)
        pltpu.sync_copy(post.at[d], posr_v)
        pltpu.async_copy(xt.at[0], ib0, si0)

        def sbody(k, _):
            for q in range(4):
                s = 4 * k + q
                p = q % 2
                ibuf, obuf = ibufs[p], obufs[q]
                pltpu.make_async_copy(xt.at[s], ibuf, sis[p]).wait()
                @pl.when(s + 1 < SEQ)
                def _():
                    pltpu.async_copy(xt.at[s + 1], ibufs[1 - p], sis[1 - p])

                @pl.when(k > 0)
                def _():
                    pltpu.make_async_copy(
                        obuf, out_slice(s, dt, di), sos[q]
                    ).wait()

                qp = (q - 1) % 4

                def _ship():
                    pltpu.async_copy(
                        obufs[qp], out_slice(s - 1, dt, di), sos[qp]
                    )

                if q == 0:
                    pl.when(k > 0)(_ship)
                else:
                    _ship()

                pv = plsc.load_gather(posr_v, [_splat(s)])

                @plsc.parallel_loop(0, GROUPS // G_IN, 1, unroll=8)
                def _(go):
                    base = go * (G_IN * L)
                    for gi in range(G_IN):
                        iv = ibuf[pl.ds(base + gi * L, L)]
                        gv = plsc.load_gather(row_v, [iv])
                        obuf[go, 0, pl.ds(gi * L, L)] = gv + pv

            return ()

        lax.fori_loop(0, SEQ // 4, sbody, ())
        for q in range(3):
            pltpu.make_async_copy(
                obufs[q], out_slice(0, 0, 0), sos[q]
            ).wait()
        pltpu.async_copy(obufs[3], out_slice(SEQ - 1, dt, di), sos[3])
        pltpu.make_async_copy(obufs[3], out_slice(0, 0, 0), sos[3]).wait()
        return ()

    lax.fori_loop(0, D_PER_W, phase, ())


@jax.jit
def kernel(x, word_table, pos_table):
    xt = x.T
    wt = word_table.T
    post = pos_table.T
    mesh = plsc.VectorSubcoreMesh(core_axis_name="c", subcore_axis_name="s")
    out5 = pl.kernel(
        _body,
        out_type=jax.ShapeDtypeStruct((SEQ, 8, 32, 8, 128), jnp.float32),
        mesh=mesh,
        compiler_params=pltpu.CompilerParams(
            use_tc_tiling_on_sc=False, needs_layout_passes=False
        ),
        scratch_types=[
            pltpu.VMEM((VOCAB,), jnp.float32),
            pltpu.VMEM((SEQ,), jnp.float32),
            pltpu.VMEM((BATCH,), jnp.int32),
            pltpu.VMEM((BATCH,), jnp.int32),
            pltpu.VMEM((32, 1, 128), jnp.float32),
            pltpu.VMEM((32, 1, 128), jnp.float32),
            pltpu.VMEM((32, 1, 128), jnp.float32),
            pltpu.VMEM((32, 1, 128), jnp.float32),
            pltpu.SemaphoreType.DMA,
            pltpu.SemaphoreType.DMA,
            pltpu.SemaphoreType.DMA,
            pltpu.SemaphoreType.DMA,
            pltpu.SemaphoreType.DMA,
            pltpu.SemaphoreType.DMA,
        ],
    )(xt, wt, post)
    return out5.transpose(2, 4, 0, 1, 3).reshape(BATCH, SEQ, D)


# speedup vs baseline: 1.0010x; 1.0000x over previous
import jax
import jax.numpy as jnp
from jax import lax
from jax.experimental import pallas as pl
from jax.experimental.pallas import tpu as pltpu
from jax.experimental.pallas import tpu_sc as plsc

NC = 2
NS = 16
NW = NC * NS
L = 16

VOCAB = 100000
D = 64
SEQ = 200
BATCH = 4096

D_PER_W = D // NW
GROUPS = BATCH // L
G_IN = 8


def _splat(val):
    return jax.lax.broadcast_in_dim(val, (L,), ())


def _body(xt, wt, post, out, row_v, posr_v, ib0, ib1, ob0, ob1, ob2, ob3,
          si0, si1, so0, so1, so2, so3):
    w = lax.axis_index("s") * NC + lax.axis_index("c")
    ibufs, sis = (ib0, ib1), (si0, si1)
    obufs, sos = (ob0, ob1, ob2, ob3), (so0, so1, so2, so3)

    def out_slice(s, dt, di):
        return out.at[s, dt, pl.ds(0, 32), pl.ds(di, 1), pl.ds(0, 128)]

    def phase(dn, _):
        d = w * D_PER_W + dn
        dt = d // 8
        di = d % 8
        pltpu.sync_copy(wt.at[d], row_v)
        pltpu.sync_copy(post.at[d], posr_v)
        pltpu.async_copy(xt.at[0], ib0, si0)

        def sbody(k, _):
            for q in range(4):
                s = 4 * k + q
                p = q % 2
                ibuf, obuf = ibufs[p], obufs[q]
                pltpu.make_async_copy(xt.at[s], ibuf, sis[p]).wait()
                @pl.when(s + 1 < SEQ)
                def _():
                    pltpu.async_copy(xt.at[s + 1], ibufs[1 - p], sis[1 - p])

                @pl.when(k > 0)
                def _():
                    pltpu.make_async_copy(
                        obuf, out_slice(s, dt, di), sos[q]
                    ).wait()

                qp = (q - 1) % 4

                def _ship():
                    pltpu.async_copy(
                        obufs[qp], out_slice(s - 1, dt, di), sos[qp]
                    )

                if q == 0:
                    pl.when(k > 0)(_ship)
                else:
                    _ship()

                pv = plsc.load_gather(posr_v, [_splat(s)])

                @plsc.parallel_loop(0, GROUPS // G_IN, 1, unroll=8)
                def _(go):
                    base = go * (G_IN * L)
                    for gi in range(G_IN):
                        iv = ibuf[pl.ds(base + gi * L, L)]
                        gv = plsc.load_gather(row_v, [iv])
                        obuf[go, 0, pl.ds(gi * L, L)] = gv + pv

            return ()

        lax.fori_loop(0, SEQ // 4, sbody, ())
        for q in range(3):
            pltpu.make_async_copy(
                obufs[q], out_slice(0, 0, 0), sos[q]
            ).wait()
        pltpu.async_copy(obufs[3], out_slice(SEQ - 1, dt, di), sos[3])
        pltpu.make_async_copy(obufs[3], out_slice(0, 0, 0), sos[3]).wait()
        return ()

    lax.fori_loop(0, D_PER_W, phase, ())


@jax.jit
def kernel(x, word_table, pos_table):
    xt = x.T
    wt = word_table.T
    post = pos_table.T
    mesh = plsc.VectorSubcoreMesh(core_axis_name="c", subcore_axis_name="s")
    out5 = pl.kernel(
        _body,
        out_type=jax.ShapeDtypeStruct((SEQ, 8, 32, 8, 128), jnp.float32),
        mesh=mesh,
        compiler_params=pltpu.CompilerParams(
            use_tc_tiling_on_sc=False, needs_layout_passes=False
        ),
        scratch_types=[
            pltpu.VMEM((VOCAB,), jnp.float32),
            pltpu.VMEM((SEQ,), jnp.float32),
            pltpu.VMEM((BATCH,), jnp.int32),
            pltpu.VMEM((BATCH,), jnp.int32),
            pltpu.VMEM((32, 1, 128), jnp.float32),
            pltpu.VMEM((32, 1, 128), jnp.float32),
            pltpu.VMEM((32, 1, 128), jnp.float32),
            pltpu.VMEM((32, 1, 128), jnp.float32),
            pltpu.SemaphoreType.DMA,
            pltpu.SemaphoreType.DMA,
            pltpu.SemaphoreType.DMA,
            pltpu.SemaphoreType.DMA,
            pltpu.SemaphoreType.DMA,
            pltpu.SemaphoreType.DMA,
        ],
    )(xt, wt, post)
    return out5.transpose(2, 4, 0, 1, 3).reshape(BATCH, SEQ, D)
